# Initial kernel scaffold; baseline (speedup 1.0000x reference)
import jax
import jax.numpy as jnp
from jax.experimental import pallas as pl


def kernel(x, edge_index, edge_weight, theta, gc_bias, fc_W, fc_b):
    raise NotImplementedError("write your pallas kernel here")


# speedup vs baseline: 5.6535x; 5.6535x over previous
import functools

import jax
import jax.numpy as jnp
from jax import lax
from jax.experimental import pallas as pl
from jax.experimental.pallas import tpu as pltpu
from jax.experimental.pallas import tpu_sc as plsc

_L = 16


def _cheb_stack_sc(x2, packed, w, K, NP):
    B, N = x2.shape
    E = packed.shape[0]
    CE = 8000
    NCH = E // CE
    NPAIR = NCH // 2
    NV = NP // _L
    NEV = CE // _L

    mesh = plsc.VectorSubcoreMesh(core_axis_name="c", subcore_axis_name="s")

    @functools.partial(
        pl.kernel,
        out_type=jax.ShapeDtypeStruct((K, B, NP), jnp.float32),
        mesh=mesh,
        compiler_params=pltpu.CompilerParams(needs_layout_passes=False,
                                             use_tc_tiling_on_sc=False),
        scratch_types=[
            pltpu.VMEM((NP,), jnp.float32),
            pltpu.VMEM((NP,), jnp.float32),
            pltpu.VMEM((NP,), jnp.float32),
            pltpu.VMEM((CE,), jnp.int32),
            pltpu.VMEM((CE,), jnp.int32),
            pltpu.VMEM((CE,), jnp.float32),
            pltpu.VMEM((CE,), jnp.float32),
            pltpu.SemaphoreType.DMA,
            pltpu.SemaphoreType.DMA,
        ],
    )
    def sck(x_hbm, pk_hbm, w_hbm, t_hbm, bA, bB, bC, pk0, pk1, w0, w1, sp, sw):
        b = lax.axis_index("s") * 2 + lax.axis_index("c")
        zeros = jnp.zeros((_L,), jnp.float32)

        def zero_full(ref):
            def body(i, c):
                ref[pl.ds(i * _L, _L)] = zeros
                return c
            lax.fori_loop(0, NV, body, 0)

        def zero_tail(ref):
            for j in range(N // _L, NV):
                ref[pl.ds(j * _L, _L)] = zeros

        def start_load(ci, pkbuf, wbuf):
            pltpu.async_copy(pk_hbm.at[pl.ds(ci * CE, CE)], pkbuf, sp)
            pltpu.async_copy(w_hbm.at[pl.ds(ci * CE, CE)], wbuf, sw)

        def wait_load(pkbuf, wbuf):
            pltpu.make_async_copy(pk_hbm.at[pl.ds(0, CE)], pkbuf, sp).wait()
            pltpu.make_async_copy(w_hbm.at[pl.ds(0, CE)], wbuf, sw).wait()

        def process(pkbuf, wbuf, cur, acc):
            def body(i, c):
                s = pl.ds(i * _L, _L)
                pk = pkbuf[s]
                wv = wbuf[s]
                srcv = jnp.bitwise_and(pk, 16383)
                dstv = jnp.right_shift(pk, 14)
                g = plsc.load_gather(cur, [srcv])
                plsc.addupdate_scatter(acc, [dstv], g * wv)
                return c
            lax.fori_loop(0, NEV, body, 0)

        def hop(cur, acc):
            start_load(0, pk0, w0)

            def pair(cp, c):
                wait_load(pk0, w0)
                start_load(2 * cp + 1, pk1, w1)
                process(pk0, w0, cur, acc)
                wait_load(pk1, w1)

                @pl.when(cp < NPAIR - 1)
                def _():
                    start_load(2 * cp + 2, pk0, w0)

                process(pk1, w1, cur, acc)
                return c
            lax.fori_loop(0, NPAIR, pair, 0)

        def cheb_update(dst, acc):
            def body(i, c):
                s = pl.ds(i * _L, _L)
                dst[s] = 2.0 * acc[s] - dst[s]
                return c
            lax.fori_loop(0, NV, body, 0)

        def emit(ref, k):
            pltpu.sync_copy(ref, t_hbm.at[k, b])

        pltpu.sync_copy(x_hbm.at[b], bA.at[pl.ds(0, N)])
        zero_tail(bA)
        emit(bA, 0)
        zero_full(bC)
        hop(bA, bC)
        emit(bC, 1)

        def dhop(kk, c):
            k0 = 2 * kk + 2
            zero_full(bB)
            hop(bC, bB)
            cheb_update(bA, bB)
            emit(bA, k0)
            zero_full(bB)
            hop(bA, bB)
            cheb_update(bC, bB)
            emit(bC, k0 + 1)
            return c
        lax.fori_loop(0, (K - 2) // 2, dhop, 0)

    return sck(x2, packed, w)


def _head_tc(t_pad, theta2, gcb, w3t, fcb, NP):
    K, B, _ = t_pad.shape
    O, C, _ = w3t.shape
    Nb = 2048
    NB = NP // Nb

    def body(theta_s, gcb_s, fcb_ref, t_ref, w_ref, out_ref, acc_ref):
        i = pl.program_id(0)

        @pl.when(i == 0)
        def _():
            acc_ref[...] = jnp.zeros_like(acc_ref)

        h = None
        for o in range(O):
            z = t_ref[0] * theta_s[0, o]
            for kk in range(1, K):
                z = z + t_ref[kk] * theta_s[kk, o]
            zo = jnp.maximum(z + gcb_s[o], 0.0)
            wo = w_ref[o]
            d = lax.dot_general(zo, wo, (((1,), (1,)), ((), ())),
                                preferred_element_type=jnp.float32)
            h = d if h is None else h + d
        acc_ref[...] += h

        @pl.when(i == NB - 1)
        def _():
            hf = jnp.maximum(acc_ref[...] + fcb_ref[...], 0.0)
            m = jnp.max(hf, axis=1, keepdims=True)
            e = jnp.exp(hf - m)
            out_ref[...] = e / jnp.sum(e, axis=1, keepdims=True)

    return pl.pallas_call(
        body,
        grid=(NB,),
        in_specs=[
            pl.BlockSpec(memory_space=pltpu.SMEM),
            pl.BlockSpec(memory_space=pltpu.SMEM),
            pl.BlockSpec((1, C), lambda i: (0, 0)),
            pl.BlockSpec((K, B, Nb), lambda i: (0, 0, i)),
            pl.BlockSpec((O, C, Nb), lambda i: (0, 0, i)),
        ],
        out_specs=pl.BlockSpec((B, C), lambda i: (0, 0)),
        out_shape=jax.ShapeDtypeStruct((B, C), jnp.float32),
        scratch_shapes=[pltpu.VMEM((B, C), jnp.float32)],
    )(theta2, gcb, fcb, t_pad, w3t)


def kernel(x, edge_index, edge_weight, theta, gc_bias, fc_W, fc_b):
    B, N, _ = x.shape
    K = theta.shape[0]
    O = theta.shape[2]
    C = fc_W.shape[1]
    NP = ((N + 2047) // 2048) * 2048

    x2 = x[:, :, 0]
    src = edge_index[0]
    dst = edge_index[1]
    packed = jnp.bitwise_or(src, jnp.left_shift(dst, 14))

    t_pad = _cheb_stack_sc(x2, packed, edge_weight, K, NP)

    theta2 = theta[:, 0, :]
    w3t = jnp.pad(fc_W.reshape(N, O, C).transpose(1, 2, 0),
                  ((0, 0), (0, 0), (0, NP - N)))
    return _head_tc(t_pad, theta2, gc_bias, w3t, fc_b.reshape(1, C), NP)


# speedup vs baseline: 17.9732x; 3.1791x over previous
import functools

import jax
import jax.numpy as jnp
from jax import lax
from jax.experimental import pallas as pl
from jax.experimental.pallas import tpu as pltpu
from jax.experimental.pallas import tpu_sc as plsc

_L = 16


def _cheb_stack_sc(x2, packed, w, K, NP):
    B, N = x2.shape
    E = packed.shape[0]
    CE = 8000
    NCH = E // CE
    NPAIR = NCH // 2
    NV = NP // _L
    NEV = CE // _L

    mesh = plsc.VectorSubcoreMesh(core_axis_name="c", subcore_axis_name="s")

    @functools.partial(
        pl.kernel,
        out_type=jax.ShapeDtypeStruct((K, B, NP), jnp.float32),
        mesh=mesh,
        compiler_params=pltpu.CompilerParams(needs_layout_passes=False,
                                             use_tc_tiling_on_sc=False),
        scratch_types=[
            pltpu.VMEM((NP,), jnp.float32),
            pltpu.VMEM((NP,), jnp.float32),
            pltpu.VMEM((NP,), jnp.float32),
            pltpu.VMEM((CE,), jnp.int32),
            pltpu.VMEM((CE,), jnp.int32),
            pltpu.VMEM((CE,), jnp.float32),
            pltpu.VMEM((CE,), jnp.float32),
            pltpu.SemaphoreType.DMA,
            pltpu.SemaphoreType.DMA,
        ],
    )
    def sck(x_hbm, pk_hbm, w_hbm, t_hbm, bA, bB, bC, pk0, pk1, w0, w1, sp, sw):
        b = lax.axis_index("s") * 2 + lax.axis_index("c")
        zeros = jnp.zeros((_L,), jnp.float32)

        def zero_full(ref):
            @plsc.parallel_loop(0, NV, 1, unroll=8)
            def _(i):
                ref[pl.ds(i * _L, _L)] = zeros

        def zero_tail(ref):
            for j in range(N // _L, NV):
                ref[pl.ds(j * _L, _L)] = zeros

        def start_load(ci, pkbuf, wbuf):
            pltpu.async_copy(pk_hbm.at[pl.ds(ci * CE, CE)], pkbuf, sp)
            pltpu.async_copy(w_hbm.at[pl.ds(ci * CE, CE)], wbuf, sw)

        def wait_load(pkbuf, wbuf):
            pltpu.make_async_copy(pk_hbm.at[pl.ds(0, CE)], pkbuf, sp).wait()
            pltpu.make_async_copy(w_hbm.at[pl.ds(0, CE)], wbuf, sw).wait()

        def process(pkbuf, wbuf, cur, acc):
            @plsc.parallel_loop(0, NEV, 1, unroll=8)
            def _(i):
                s = pl.ds(i * _L, _L)
                pk = pkbuf[s]
                wv = wbuf[s]
                srcv = jnp.bitwise_and(pk, 16383)
                dstv = jnp.right_shift(pk, 14)
                g = plsc.load_gather(cur, [srcv])
                plsc.addupdate_scatter(acc, [dstv], g * wv)

        def hop(cur, acc):
            start_load(0, pk0, w0)

            def pair(cp, c):
                wait_load(pk0, w0)
                start_load(2 * cp + 1, pk1, w1)
                process(pk0, w0, cur, acc)
                wait_load(pk1, w1)

                @pl.when(cp < NPAIR - 1)
                def _():
                    start_load(2 * cp + 2, pk0, w0)

                process(pk1, w1, cur, acc)
                return c
            lax.fori_loop(0, NPAIR, pair, 0)

        def cheb_update(dst, acc):
            @plsc.parallel_loop(0, NV, 1, unroll=8)
            def _(i):
                s = pl.ds(i * _L, _L)
                dst[s] = 2.0 * acc[s] - dst[s]

        def emit(ref, k):
            pltpu.sync_copy(ref, t_hbm.at[k, b])

        pltpu.sync_copy(x_hbm.at[b], bA.at[pl.ds(0, N)])
        zero_tail(bA)
        emit(bA, 0)
        zero_full(bC)
        hop(bA, bC)
        emit(bC, 1)

        def dhop(kk, c):
            k0 = 2 * kk + 2
            zero_full(bB)
            hop(bC, bB)
            cheb_update(bA, bB)
            emit(bA, k0)
            zero_full(bB)
            hop(bA, bB)
            cheb_update(bC, bB)
            emit(bC, k0 + 1)
            return c
        lax.fori_loop(0, (K - 2) // 2, dhop, 0)

    return sck(x2, packed, w)


def _head_tc(t_pad, theta2, gcb, w3t, fcb, NP):
    K, B, _ = t_pad.shape
    O, C, _ = w3t.shape
    Nb = 2048
    NB = NP // Nb

    def body(theta_s, gcb_s, fcb_ref, t_ref, w_ref, out_ref, acc_ref):
        i = pl.program_id(0)

        @pl.when(i == 0)
        def _():
            acc_ref[...] = jnp.zeros_like(acc_ref)

        h = None
        for o in range(O):
            z = t_ref[0] * theta_s[0, o]
            for kk in range(1, K):
                z = z + t_ref[kk] * theta_s[kk, o]
            zo = jnp.maximum(z + gcb_s[o], 0.0)
            wo = w_ref[o]
            d = lax.dot_general(zo, wo, (((1,), (1,)), ((), ())),
                                preferred_element_type=jnp.float32)
            h = d if h is None else h + d
        acc_ref[...] += h

        @pl.when(i == NB - 1)
        def _():
            hf = jnp.maximum(acc_ref[...] + fcb_ref[...], 0.0)
            m = jnp.max(hf, axis=1, keepdims=True)
            e = jnp.exp(hf - m)
            out_ref[...] = e / jnp.sum(e, axis=1, keepdims=True)

    return pl.pallas_call(
        body,
        grid=(NB,),
        in_specs=[
            pl.BlockSpec(memory_space=pltpu.SMEM),
            pl.BlockSpec(memory_space=pltpu.SMEM),
            pl.BlockSpec((1, C), lambda i: (0, 0)),
            pl.BlockSpec((K, B, Nb), lambda i: (0, 0, i)),
            pl.BlockSpec((O, C, Nb), lambda i: (0, 0, i)),
        ],
        out_specs=pl.BlockSpec((B, C), lambda i: (0, 0)),
        out_shape=jax.ShapeDtypeStruct((B, C), jnp.float32),
        scratch_shapes=[pltpu.VMEM((B, C), jnp.float32)],
    )(theta2, gcb, fcb, t_pad, w3t)


def kernel(x, edge_index, edge_weight, theta, gc_bias, fc_W, fc_b):
    B, N, _ = x.shape
    K = theta.shape[0]
    O = theta.shape[2]
    C = fc_W.shape[1]
    NP = ((N + 2047) // 2048) * 2048

    x2 = x[:, :, 0]
    src = edge_index[0]
    dst = edge_index[1]
    packed = jnp.bitwise_or(src, jnp.left_shift(dst, 14))

    t_pad = _cheb_stack_sc(x2, packed, edge_weight, K, NP)

    theta2 = theta[:, 0, :]
    w3t = jnp.pad(fc_W.reshape(N, O, C).transpose(1, 2, 0),
                  ((0, 0), (0, 0), (0, NP - N)))
    return _head_tc(t_pad, theta2, gc_bias, w3t, fc_b.reshape(1, C), NP)


# speedup vs baseline: 18.5533x; 1.0323x over previous
import functools

import jax
import jax.numpy as jnp
from jax import lax
from jax.experimental import pallas as pl
from jax.experimental.pallas import tpu as pltpu
from jax.experimental.pallas import tpu_sc as plsc

_L = 16


def _cheb_stack_sc(x2, packed, w, K, NP):
    B, N = x2.shape
    E = packed.shape[0]
    CE = 8000
    NCH = E // CE
    NPAIR = NCH // 2
    NV = NP // _L
    NEV = CE // _L

    mesh = plsc.VectorSubcoreMesh(core_axis_name="c", subcore_axis_name="s")

    @functools.partial(
        pl.kernel,
        out_type=jax.ShapeDtypeStruct((K, B, NP), jnp.float32),
        mesh=mesh,
        compiler_params=pltpu.CompilerParams(needs_layout_passes=False,
                                             use_tc_tiling_on_sc=False),
        scratch_types=[
            pltpu.VMEM((NP,), jnp.float32),
            pltpu.VMEM((NP,), jnp.float32),
            pltpu.VMEM((NP,), jnp.float32),
            pltpu.VMEM((CE,), jnp.int32),
            pltpu.VMEM((CE,), jnp.int32),
            pltpu.VMEM((CE,), jnp.float32),
            pltpu.VMEM((CE,), jnp.float32),
            pltpu.SemaphoreType.DMA,
            pltpu.SemaphoreType.DMA,
            pltpu.SemaphoreType.DMA,
        ],
    )
    def sck(x_hbm, pk_hbm, w_hbm, t_hbm, bA, bB, bC, pk0, pk1, w0, w1, sp, sw,
            se):
        b = lax.axis_index("s") * 2 + lax.axis_index("c")
        zeros = jnp.zeros((_L,), jnp.float32)

        def zero_full(ref):
            @plsc.parallel_loop(0, NV, 1, unroll=8)
            def _(i):
                ref[pl.ds(i * _L, _L)] = zeros

        def zero_tail(ref):
            for j in range(N // _L, NV):
                ref[pl.ds(j * _L, _L)] = zeros

        def start_load(ci, pkbuf, wbuf):
            pltpu.async_copy(pk_hbm.at[pl.ds(ci * CE, CE)], pkbuf, sp)
            pltpu.async_copy(w_hbm.at[pl.ds(ci * CE, CE)], wbuf, sw)

        def wait_load(pkbuf, wbuf):
            pltpu.make_async_copy(pk_hbm.at[pl.ds(0, CE)], pkbuf, sp).wait()
            pltpu.make_async_copy(w_hbm.at[pl.ds(0, CE)], wbuf, sw).wait()

        def process(pkbuf, wbuf, cur, acc):
            @plsc.parallel_loop(0, NEV, 1, unroll=8)
            def _(i):
                s = pl.ds(i * _L, _L)
                pk = pkbuf[s]
                wv = wbuf[s]
                srcv = jnp.bitwise_and(pk, 16383)
                dstv = jnp.right_shift(pk, 14)
                g = plsc.load_gather(cur, [srcv])
                plsc.addupdate_scatter(acc, [dstv], g * wv)

        def hop(cur, acc):
            def pair(cp, c):
                wait_load(pk0, w0)
                start_load(2 * cp + 1, pk1, w1)
                process(pk0, w0, cur, acc)
                wait_load(pk1, w1)

                @pl.when(cp < NPAIR - 1)
                def _():
                    start_load(2 * cp + 2, pk0, w0)

                process(pk1, w1, cur, acc)
                return c
            lax.fori_loop(0, NPAIR, pair, 0)

        def cheb_fuse(dst, acc):
            @plsc.parallel_loop(0, NV, 1, unroll=8)
            def _(i):
                s = pl.ds(i * _L, _L)
                dst[s] = 2.0 * acc[s] - dst[s]
                acc[s] = zeros

        def emit(ref, k):
            pltpu.async_copy(ref, t_hbm.at[k, b], se)

        def wait_emit():
            pltpu.make_async_copy(bA, t_hbm.at[0, b], se).wait()

        NDH = (K - 2) // 2

        pltpu.sync_copy(x_hbm.at[b], bA.at[pl.ds(0, N)])
        zero_tail(bA)
        start_load(0, pk0, w0)
        emit(bA, 0)
        zero_full(bC)
        zero_full(bB)
        hop(bA, bC)
        start_load(0, pk0, w0)
        emit(bC, 1)

        def dhop(kk, c):
            k0 = 2 * kk + 2
            hop(bC, bB)
            start_load(0, pk0, w0)
            wait_emit()
            cheb_fuse(bA, bB)
            emit(bA, k0)
            hop(bA, bB)

            @pl.when(kk < NDH - 1)
            def _():
                start_load(0, pk0, w0)

            wait_emit()
            cheb_fuse(bC, bB)
            emit(bC, k0 + 1)
            return c
        lax.fori_loop(0, NDH, dhop, 0)
        wait_emit()
        wait_emit()

    return sck(x2, packed, w)


def _head_tc(t_pad, theta2, gcb, w3t, fcb, NP):
    K, B, _ = t_pad.shape
    O, C, _ = w3t.shape
    Nb = 2048
    NB = NP // Nb

    def body(theta_s, gcb_s, fcb_ref, t_ref, w_ref, out_ref, acc_ref):
        i = pl.program_id(0)

        @pl.when(i == 0)
        def _():
            acc_ref[...] = jnp.zeros_like(acc_ref)

        h = None
        for o in range(O):
            z = t_ref[0] * theta_s[0, o]
            for kk in range(1, K):
                z = z + t_ref[kk] * theta_s[kk, o]
            zo = jnp.maximum(z + gcb_s[o], 0.0)
            wo = w_ref[o]
            d = lax.dot_general(zo, wo, (((1,), (1,)), ((), ())),
                                preferred_element_type=jnp.float32)
            h = d if h is None else h + d
        acc_ref[...] += h

        @pl.when(i == NB - 1)
        def _():
            hf = jnp.maximum(acc_ref[...] + fcb_ref[...], 0.0)
            m = jnp.max(hf, axis=1, keepdims=True)
            e = jnp.exp(hf - m)
            out_ref[...] = e / jnp.sum(e, axis=1, keepdims=True)

    return pl.pallas_call(
        body,
        grid=(NB,),
        in_specs=[
            pl.BlockSpec(memory_space=pltpu.SMEM),
            pl.BlockSpec(memory_space=pltpu.SMEM),
            pl.BlockSpec((1, C), lambda i: (0, 0)),
            pl.BlockSpec((K, B, Nb), lambda i: (0, 0, i)),
            pl.BlockSpec((O, C, Nb), lambda i: (0, 0, i)),
        ],
        out_specs=pl.BlockSpec((B, C), lambda i: (0, 0)),
        out_shape=jax.ShapeDtypeStruct((B, C), jnp.float32),
        scratch_shapes=[pltpu.VMEM((B, C), jnp.float32)],
    )(theta2, gcb, fcb, t_pad, w3t)


def kernel(x, edge_index, edge_weight, theta, gc_bias, fc_W, fc_b):
    B, N, _ = x.shape
    K = theta.shape[0]
    O = theta.shape[2]
    C = fc_W.shape[1]
    NP = ((N + 2047) // 2048) * 2048

    x2 = x[:, :, 0]
    src = edge_index[0]
    dst = edge_index[1]
    packed = jnp.bitwise_or(src, jnp.left_shift(dst, 14))

    t_pad = _cheb_stack_sc(x2, packed, edge_weight, K, NP)

    theta2 = theta[:, 0, :]
    w3t = jnp.pad(fc_W.reshape(N, O, C).transpose(1, 2, 0),
                  ((0, 0), (0, 0), (0, NP - N)))
    return _head_tc(t_pad, theta2, gc_bias, w3t, fc_b.reshape(1, C), NP)
